# Initial kernel scaffold; baseline (speedup 1.0000x reference)
import jax
import jax.numpy as jnp
from jax.experimental import pallas as pl


def kernel(node_features, adjacency_matrix, W_self, b_self, W_nb, b_nb, W_att, b_att, ln_gamma, ln_beta):
    raise NotImplementedError("write your pallas kernel here")


# speedup vs baseline: 4.5474x; 4.5474x over previous
import functools
import jax
import jax.numpy as jnp
from jax.experimental import pallas as pl
from jax.experimental.pallas import tpu as pltpu

_BLK = 256


def _fused_body(x_ref, a_ref, wcat_ref, bcat_ref, w2_ref, gamma_ref, beta_ref,
                out_ref, m_scr, self_scr):
    i = pl.program_id(0)
    B, N, F = x_ref.shape

    @pl.when(i == 0)
    def _prep():
        es = []
        for b in range(B):
            x = x_ref[b]
            h = jnp.dot(x, wcat_ref[...], preferred_element_type=jnp.float32)
            h = h + bcat_ref[...]
            self_scr[b] = h[:, :F]
            nb = h[:, F:]
            att = jnp.sum(x * w2_ref[...], axis=1, keepdims=True)
            e = jnp.exp(att - jnp.max(att))
            m_scr[:, b * F:(b + 1) * F] = e * nb
            es.append(e)
        es.append(jnp.zeros((N, F - B), dtype=jnp.float32))
        m_scr[:, B * F:] = jnp.concatenate(es, axis=1)

    mm = jnp.dot(a_ref[...], m_scr[...], preferred_element_type=jnp.float32)
    for b in range(B):
        num = mm[:, b * F:(b + 1) * F]
        den = mm[:, B * F + b:B * F + b + 1]
        agg = jnp.where(den > 0, num / den, 0.0)
        comb = self_scr[b, pl.ds(i * _BLK, _BLK), :] + agg
        mean = jnp.mean(comb, axis=1, keepdims=True)
        cent = comb - mean
        var = jnp.mean(cent * cent, axis=1, keepdims=True)
        normed = cent / jnp.sqrt(var + 1e-5)
        out_ref[b] = jnp.maximum(normed * gamma_ref[...] + beta_ref[...], 0.0)


def kernel(node_features, adjacency_matrix, W_self, b_self, W_nb, b_nb,
           W_att, b_att, ln_gamma, ln_beta):
    B, N, F = node_features.shape
    w2 = W_att[F:].reshape(1, F)
    w_cat = jnp.concatenate([W_self, W_nb], axis=1)
    b_cat = jnp.concatenate([b_self, b_nb]).reshape(1, 2 * F)
    gamma = ln_gamma.reshape(1, F)
    beta = ln_beta.reshape(1, F)

    grid = (N // _BLK,)
    out = pl.pallas_call(
        _fused_body,
        grid=grid,
        in_specs=[
            pl.BlockSpec((B, N, F), lambda i: (0, 0, 0)),
            pl.BlockSpec((_BLK, N), lambda i: (i, 0)),
            pl.BlockSpec((F, 2 * F), lambda i: (0, 0)),
            pl.BlockSpec((1, 2 * F), lambda i: (0, 0)),
            pl.BlockSpec((1, F), lambda i: (0, 0)),
            pl.BlockSpec((1, F), lambda i: (0, 0)),
            pl.BlockSpec((1, F), lambda i: (0, 0)),
        ],
        out_specs=pl.BlockSpec((B, _BLK, F), lambda i: (0, i, 0)),
        out_shape=jax.ShapeDtypeStruct((B, N, F), jnp.float32),
        scratch_shapes=[
            pltpu.VMEM((N, (B + 1) * F), jnp.float32),
            pltpu.VMEM((B, N, F), jnp.float32),
        ],
        compiler_params=pltpu.CompilerParams(
            dimension_semantics=("arbitrary",),
        ),
    )(node_features, adjacency_matrix, w_cat, b_cat, w2, gamma, beta)
    return out
